# Initial kernel scaffold; baseline (speedup 1.0000x reference)
import jax
import jax.numpy as jnp
from jax.experimental import pallas as pl


def kernel(indices, table):
    raise NotImplementedError("write your pallas kernel here")


# speedup vs baseline: 1.8389x; 1.8389x over previous
import functools

import jax
import jax.numpy as jnp
from jax import lax
from jax.experimental import pallas as pl
from jax.experimental.pallas import tpu as pltpu
from jax.experimental.pallas import tpu_sc as plsc

EMBED_DIM = 64
CHUNK = 128

_info = plsc.get_sparse_core_info()
_NW = _info.num_cores * _info.num_subcores


@functools.lru_cache(maxsize=None)
def _build(B: int, D: int):
    b_per_w = B // _NW
    n_chunks = b_per_w // CHUNK
    assert B % (_NW * CHUNK) == 0 and n_chunks % 2 == 0

    mesh = plsc.VectorSubcoreMesh(core_axis_name="c", subcore_axis_name="s")

    @functools.partial(
        pl.kernel,
        mesh=mesh,
        compiler_params=pltpu.CompilerParams(use_tc_tiling_on_sc=False),
        out_type=jax.ShapeDtypeStruct((B, D), jnp.float32),
        scratch_types=[
            pltpu.VMEM((b_per_w,), jnp.int32),
            pltpu.VMEM((CHUNK, D), jnp.float32),
            pltpu.VMEM((CHUNK, D), jnp.float32),
            pltpu.SemaphoreType.DMA,
            pltpu.SemaphoreType.DMA,
        ],
    )
    def gather_kernel(idx_hbm, table_hbm, out_hbm, idx_v, row0, row1, sem0, sem1):
        wid = lax.axis_index("s") * _info.num_cores + lax.axis_index("c")
        base = wid * b_per_w
        pltpu.sync_copy(idx_hbm.at[pl.ds(base, b_per_w)], idx_v)

        bufs = (row0, row1)
        sems = (sem0, sem1)

        def gather(g, b):
            return pltpu.make_async_copy(
                table_hbm.at[idx_v.at[pl.ds(g * CHUNK, CHUNK)]], bufs[b], sems[b]
            )

        gather(0, 0).start()
        gather(1, 1).start()

        def pair(i, carry):
            for b in range(2):
                g = 2 * i + b
                gather(g, b).wait()
                pltpu.sync_copy(bufs[b], out_hbm.at[pl.ds(base + g * CHUNK, CHUNK)])

                @pl.when(g + 2 < n_chunks)
                def _():
                    gather(g + 2, b).start()

            return carry

        lax.fori_loop(0, n_chunks // 2, pair, 0)

    return gather_kernel


def kernel(indices, table):
    bsz, hist = indices.shape
    flat = indices.reshape(bsz * hist).astype(jnp.int32)
    out = _build(bsz * hist, table.shape[1])(flat, table)
    return out.reshape(bsz, hist, table.shape[1])


# speedup vs baseline: 1.8724x; 1.0182x over previous
import functools

import jax
import jax.numpy as jnp
from jax import lax
from jax.experimental import pallas as pl
from jax.experimental.pallas import tpu as pltpu
from jax.experimental.pallas import tpu_sc as plsc

EMBED_DIM = 64
CHUNK = 512

_info = plsc.get_sparse_core_info()
_NW = _info.num_cores * _info.num_subcores


@functools.lru_cache(maxsize=None)
def _build(B: int, D: int):
    b_per_w = B // _NW
    n_chunks = b_per_w // CHUNK
    assert B % (_NW * CHUNK) == 0 and n_chunks % 2 == 0

    mesh = plsc.VectorSubcoreMesh(core_axis_name="c", subcore_axis_name="s")

    @functools.partial(
        pl.kernel,
        mesh=mesh,
        compiler_params=pltpu.CompilerParams(use_tc_tiling_on_sc=False),
        out_type=jax.ShapeDtypeStruct((B, D), jnp.float32),
        scratch_types=[
            pltpu.VMEM((b_per_w,), jnp.int32),
            pltpu.VMEM((CHUNK, D), jnp.float32),
            pltpu.VMEM((CHUNK, D), jnp.float32),
            pltpu.SemaphoreType.DMA,
            pltpu.SemaphoreType.DMA,
        ],
    )
    def gather_kernel(idx_hbm, table_hbm, out_hbm, idx_v, row0, row1, sem0, sem1):
        wid = lax.axis_index("s") * _info.num_cores + lax.axis_index("c")
        base = wid * b_per_w
        pltpu.sync_copy(idx_hbm.at[pl.ds(base, b_per_w)], idx_v)

        bufs = (row0, row1)
        sems = (sem0, sem1)

        def gather(g, b):
            return pltpu.make_async_copy(
                table_hbm.at[idx_v.at[pl.ds(g * CHUNK, CHUNK)]], bufs[b], sems[b]
            )

        gather(0, 0).start()
        gather(1, 1).start()

        def pair(i, carry):
            for b in range(2):
                g = 2 * i + b
                gather(g, b).wait()
                pltpu.sync_copy(bufs[b], out_hbm.at[pl.ds(base + g * CHUNK, CHUNK)])

                @pl.when(g + 2 < n_chunks)
                def _():
                    gather(g + 2, b).start()

            return carry

        lax.fori_loop(0, n_chunks // 2, pair, 0)

    return gather_kernel


def kernel(indices, table):
    bsz, hist = indices.shape
    flat = indices.reshape(bsz * hist).astype(jnp.int32)
    out = _build(bsz * hist, table.shape[1])(flat, table)
    return out.reshape(bsz, hist, table.shape[1])


# speedup vs baseline: 1.8776x; 1.0028x over previous
import functools

import jax
import jax.numpy as jnp
from jax import lax
from jax.experimental import pallas as pl
from jax.experimental.pallas import tpu as pltpu
from jax.experimental.pallas import tpu_sc as plsc

EMBED_DIM = 64
CHUNK = 128
NBUF = 8

_info = plsc.get_sparse_core_info()
_NW = _info.num_cores * _info.num_subcores


@functools.lru_cache(maxsize=None)
def _build(B: int, D: int):
    b_per_w = B // _NW
    n_chunks = b_per_w // CHUNK
    assert B % (_NW * CHUNK) == 0 and n_chunks % NBUF == 0

    mesh = plsc.VectorSubcoreMesh(core_axis_name="c", subcore_axis_name="s")

    @functools.partial(
        pl.kernel,
        mesh=mesh,
        compiler_params=pltpu.CompilerParams(use_tc_tiling_on_sc=False),
        out_type=jax.ShapeDtypeStruct((B, D), jnp.float32),
        scratch_types=[
            pltpu.VMEM((b_per_w,), jnp.int32),
            pltpu.VMEM((NBUF, CHUNK, D), jnp.float32),
        ]
        + [pltpu.SemaphoreType.DMA] * (2 * NBUF),
    )
    def gather_kernel(idx_hbm, table_hbm, out_hbm, idx_v, rows_v, *sems):
        gsems, ssems = sems[:NBUF], sems[NBUF:]
        wid = lax.axis_index("s") * _info.num_cores + lax.axis_index("c")
        base = wid * b_per_w
        pltpu.sync_copy(idx_hbm.at[pl.ds(base, b_per_w)], idx_v)

        def gather(g, b):
            return pltpu.make_async_copy(
                table_hbm.at[idx_v.at[pl.ds(g * CHUNK, CHUNK)]],
                rows_v.at[b],
                gsems[b],
            )

        def store(g, b):
            return pltpu.make_async_copy(
                rows_v.at[b], out_hbm.at[pl.ds(base + g * CHUNK, CHUNK)], ssems[b]
            )

        for b in range(NBUF):
            gather(b, b).start()

        def ring(i, carry):
            g0 = i * NBUF
            for b in range(NBUF):
                g = g0 + b
                gather(g, b).wait()
                store(g, b).start()
                nxt = g + NBUF - 1

                @pl.when(jnp.logical_and(g >= 1, nxt < n_chunks))
                def _():
                    bb = (b + NBUF - 1) % NBUF
                    store(g - 1, bb).wait()
                    gather(nxt, bb).start()

            return carry

        lax.fori_loop(0, n_chunks // NBUF, ring, 0)
        for k in range(NBUF):
            g = n_chunks - NBUF + k
            store(g, g % NBUF).wait()

    return gather_kernel


def kernel(indices, table):
    bsz, hist = indices.shape
    flat = indices.reshape(bsz * hist).astype(jnp.int32)
    out = _build(bsz * hist, table.shape[1])(flat, table)
    return out.reshape(bsz, hist, table.shape[1])
